# baseline (speedup 1.0000x reference)
import jax
import jax.numpy as jnp
from jax import lax
from jax.experimental import pallas as pl
from jax.experimental.pallas import tpu as pltpu
from jax.experimental.pallas import tpu_sc as plsc

_TOPK = 50
_Q, _G, _D, _C = 1024, 32768, 128, 8
_QB = 128
_GBK = 4096
_BISECT = 12

_L = 16
_NC, _NS = 2, 16
_NW = _NC * _NS
_RW = _G // _NW
_RC = 128
_NCH = _RW // _RC


def _sc_norm_body(gf_hbm, cam_hbm, gm_hbm, gs_hbm, out_hbm,
                  tab_v, cam_v, g_v, o_v):
    wid = lax.axis_index("s") * _NC + lax.axis_index("c")
    base = wid * _RW
    pltpu.sync_copy(gm_hbm, tab_v.at[pl.ds(0, _C)])
    pltpu.sync_copy(gs_hbm, tab_v.at[pl.ds(_C, _C)])
    for r in range(_C):
        for j in range(_D // _L):
            sl = pl.ds(j * _L, _L)
            tab_v[_C + r, sl] = 1.0 / tab_v[_C + r, sl]

    def chunk_body(ci, carry):
        rbase = base + ci * _RC
        pltpu.sync_copy(cam_hbm.at[pl.ds(rbase, _RC)], cam_v)
        pltpu.sync_copy(gf_hbm.at[pl.ds(rbase, _RC)], g_v)

        def grp_body(gidx, carry2):
            r0 = gidx * _L
            rows = lax.iota(jnp.int32, _L) + r0
            cams = cam_v[pl.ds(r0, _L)]
            cams_s = cams + _C

            def col_body(j, carry3):
                colb = jnp.full((_L,), j, jnp.int32)
                g16 = plsc.load_gather(g_v, [rows, colb])
                m16 = plsc.load_gather(tab_v, [cams, colb])
                i16 = plsc.load_gather(tab_v, [cams_s, colb])
                plsc.store_scatter(o_v, [rows, colb], (g16 - m16) * i16)
                return carry3

            lax.fori_loop(0, _D, col_body, 0)
            return carry2

        lax.fori_loop(0, _RC // _L, grp_body, 0)
        pltpu.sync_copy(o_v, out_hbm.at[pl.ds(rbase, _RC)])
        return carry

    lax.fori_loop(0, _NCH, chunk_body, 0)


def _dist_topk_body(x_ref, c_ref, qm_ref, qs_ref, gf_ref,
                    xn_ref, loss_ref, d2_ref, rmin_ref, rmax_ref):
    qi = pl.program_id(0)
    gi = pl.program_id(1)
    ng = pl.num_programs(1)

    @pl.when(gi == 0)
    def _():
        cam = c_ref[0]
        iot = lax.broadcasted_iota(jnp.int32, (_C, _QB), 0)
        onehot = (iot == cam).astype(jnp.float32)
        m = lax.dot_general(onehot, qm_ref[...], (((0,), (0,)), ((), ())),
                            preferred_element_type=jnp.float32)
        s = lax.dot_general(onehot, qs_ref[...], (((0,), (0,)), ((), ())),
                            preferred_element_type=jnp.float32)
        xn_ref[...] = (x_ref[...] - m) / s

    @pl.when((qi == 0) & (gi == 0))
    def _():
        loss_ref[...] = jnp.zeros_like(loss_ref)

    xn = xn_ref[...]
    gfb = gf_ref[...]
    xx = jnp.sum(xn * xn, axis=1, keepdims=True)
    ones_row = jnp.ones((1, _D), jnp.float32)
    gg = lax.dot_general(ones_row, gfb * gfb, (((1,), (1,)), ((), ())),
                         preferred_element_type=jnp.float32)
    xg = lax.dot_general(xn, gfb, (((1,), (1,)), ((), ())),
                         preferred_element_type=jnp.float32)
    d2b = jnp.maximum(xx + gg - 2.0 * xg, 1e-12)
    d2_ref[:, pl.ds(gi * _GBK, _GBK)] = d2b
    bmin = jnp.min(d2b, axis=1, keepdims=True)
    bmax = jnp.max(d2b, axis=1, keepdims=True)

    @pl.when(gi == 0)
    def _():
        rmin_ref[...] = bmin
        rmax_ref[...] = bmax

    @pl.when(gi > 0)
    def _():
        rmin_ref[...] = jnp.minimum(rmin_ref[...], bmin)
        rmax_ref[...] = jnp.maximum(rmax_ref[...], bmax)

    @pl.when(gi == ng - 1)
    def _():
        d2 = d2_ref[...]
        ones_g = jnp.ones((1, _G), jnp.float32)

        def body(_, carry):
            lo_c, hi_c = carry
            mid = 0.5 * (lo_c + hi_c)
            cnt = lax.dot_general(jnp.where(d2 <= mid, 1.0, 0.0), ones_g,
                                  (((1,), (1,)), ((), ())),
                                  preferred_element_type=jnp.float32)
            pred = cnt >= float(_TOPK)
            return (jnp.where(pred, lo_c, mid), jnp.where(pred, mid, hi_c))

        _, vk = lax.fori_loop(0, _BISECT, body,
                              (rmin_ref[...], rmax_ref[...]))
        mask = d2 < vk
        cnt_lt = jnp.sum(jnp.where(mask, 1.0, 0.0), axis=1, keepdims=True)
        ssum = jnp.sum(jnp.where(mask, jnp.sqrt(d2), 0.0), axis=1,
                       keepdims=True)
        row = ssum + (float(_TOPK) - cnt_lt) * jnp.sqrt(vk)
        loss_ref[...] += jnp.sum(row, keepdims=True) * (1.0 / float(_Q))


def kernel(x, c, gallery_feats, gallery_camids, gmeans, gstds, qmeans, qstds):
    c32 = c.astype(jnp.int32).reshape(_Q // _QB, 1, _QB)
    gc32 = gallery_camids.astype(jnp.int32)

    mesh = plsc.VectorSubcoreMesh(core_axis_name="c", subcore_axis_name="s")
    gf_norm = pl.kernel(
        _sc_norm_body,
        out_type=jax.ShapeDtypeStruct((_G, _D), jnp.float32),
        mesh=mesh,
        scratch_types=[
            pltpu.VMEM((2 * _C, _D), jnp.float32),
            pltpu.VMEM((_RC,), jnp.int32),
            pltpu.VMEM((_RC, _D), jnp.float32),
            pltpu.VMEM((_RC, _D), jnp.float32),
        ],
        compiler_params=pltpu.CompilerParams(needs_layout_passes=False),
    )(gallery_feats, gc32, gmeans, gstds)

    x_norm, loss2d = pl.pallas_call(
        _dist_topk_body,
        grid=(_Q // _QB, _G // _GBK),
        in_specs=[
            pl.BlockSpec((_QB, _D), lambda qi, gi: (qi, 0)),
            pl.BlockSpec((1, 1, _QB), lambda qi, gi: (qi, 0, 0)),
            pl.BlockSpec((_C, _D), lambda qi, gi: (0, 0)),
            pl.BlockSpec((_C, _D), lambda qi, gi: (0, 0)),
            pl.BlockSpec((_GBK, _D), lambda qi, gi: (gi, 0)),
        ],
        out_specs=[
            pl.BlockSpec((_QB, _D), lambda qi, gi: (qi, 0)),
            pl.BlockSpec((1, 1), lambda qi, gi: (0, 0)),
        ],
        out_shape=[
            jax.ShapeDtypeStruct((_Q, _D), jnp.float32),
            jax.ShapeDtypeStruct((1, 1), jnp.float32),
        ],
        scratch_shapes=[pltpu.VMEM((_QB, _G), jnp.float32),
                        pltpu.VMEM((_QB, 1), jnp.float32),
                        pltpu.VMEM((_QB, 1), jnp.float32)],
    )(x, c32, qmeans, qstds, gf_norm)

    return (x_norm, gf_norm, loss2d[0, 0])
